# baseline (speedup 1.0000x reference)
import functools

import jax
import jax.numpy as jnp
from jax.experimental import pallas as pl
from jax.experimental.pallas import tpu as pltpu

TEMPERATURE = 0.1


def _fused_kernel(starts_ref, lims_ref, ch_ref, cc_ref, codec_ref, c2c_ref,
                  nb_ref, ns_ref, sall_ref, lall_ref, out_ref, pooled_ref,
                  *, B, C, K, N, HB):
    b = pl.program_id(0)
    h = pl.program_id(1)
    nh = pl.num_programs(1)
    s = starts_ref[0, 0, :]
    lim = lims_ref[0, 0, :]
    L = ch_ref.shape[1]
    t = jax.lax.broadcasted_iota(jnp.int32, (C, L), 1)
    mask = (t >= s[:, None]) & (t <= lim[:, None])
    maskf = mask.astype(jnp.float32)
    pooled_ref[pl.ds(b * C, C), pl.ds(h * HB, HB)] = jnp.dot(
        maskf, ch_ref[0], preferred_element_type=jnp.float32)

    @pl.when((b == B - 1) & (h == nh - 1))
    def _epilogue():
        eps = jnp.float32(1e-12)
        cc = cc_ref[...]
        cc = cc / jnp.maximum(
            jnp.sqrt(jnp.sum(cc * cc, axis=1, keepdims=True)), eps)
        codec = codec_ref[...]
        codec = codec / jnp.maximum(
            jnp.sqrt(jnp.sum(codec * codec, axis=1, keepdims=True)), eps)

        c2c = c2c_ref[0, 0, :]
        c2c_cl = jnp.clip(c2c, 0, N - 1)
        jj = jax.lax.broadcasted_iota(jnp.int32, (N, N), 1)
        sel_pos = (jj == c2c_cl[:, None]).astype(jnp.float32)
        code_cent = jnp.dot(sel_pos, codec,
                            preferred_element_type=jnp.float32)
        pos_sim = jnp.sum(cc * code_cent, axis=1)

        cnt = jnp.maximum(
            lall_ref[0, 0, :] - sall_ref[0, 0, :] + 1, 0
        ).astype(jnp.float32)
        pooled = pooled_ref[...]
        pooled = pooled / jnp.maximum(cnt, 1.0)[:, None]
        pooled = pooled / jnp.maximum(
            jnp.sqrt(jnp.sum(pooled * pooled, axis=1, keepdims=True)), eps)
        S = jnp.dot(cc, pooled.T,
                    preferred_element_type=jnp.float32)

        nb = nb_ref[0, :, :]
        ns = ns_ref[0, :, :]
        in_range = (nb < B) & (ns < C)
        j = jnp.clip(nb, 0, B - 1) * C + jnp.clip(ns, 0, C - 1)
        jk = jax.lax.broadcasted_iota(jnp.int32, (N, K, N), 2)
        sel = (jk == j[:, :, None]).astype(jnp.float32)
        E = jnp.sum(S[:, None, :] * sel, axis=2)
        cnt_pos = (cnt > 0.0).astype(jnp.float32)
        neg_has = jnp.sum(cnt_pos[None, None, :] * sel, axis=2) > 0.0
        vmask = in_range & neg_has

        neg_exp = jnp.exp(E / TEMPERATURE)
        neg_sum = jnp.sum(jnp.where(vmask, neg_exp, 0.0), axis=1)
        pos_exp = jnp.exp(pos_sim / TEMPERATURE)
        lv = -jnp.log(pos_exp / (pos_exp + neg_sum + 1e-08))
        valid = (c2c < N) & jnp.any(vmask, axis=1)
        vals = jnp.where(valid, lv, 0.0)
        total = jnp.sum(vals)
        n = jnp.sum(valid.astype(jnp.float32))
        res = jnp.where(n > 0.0, total / jnp.maximum(n, 1.0), 0.0)
        out_ref[...] = jnp.reshape(res, (1, 1))


@jax.jit
def kernel(comment_centers, code_centers, all_code_centers,
           comment_to_code_map, negative_sample_indices, nl_hidden,
           code_hidden, total_code_tokens_list, valid_code_spans_batch,
           valid_comment_spans_batch, step_descriptions_batch):
    del all_code_centers, nl_hidden, valid_comment_spans_batch
    del step_descriptions_batch
    B, L, H = code_hidden.shape
    N, _ = comment_centers.shape
    _, C, K, _ = negative_sample_indices.shape

    spans = valid_code_spans_batch.astype(jnp.int32)
    starts = spans[:, :, 1, 0].reshape(B, 1, C)
    totals = total_code_tokens_list.astype(jnp.int32)
    lims = jnp.minimum(spans[:, :, 1, 1],
                       totals[:, None]).reshape(B, 1, C)

    negs = negative_sample_indices.astype(jnp.int32).reshape(N, K, 2)
    nb = negs[:, :, 0].reshape(1, N, K)
    ns = negs[:, :, 1].reshape(1, N, K)
    c2c = comment_to_code_map.astype(jnp.int32).reshape(1, 1, N)

    HSPLIT = 2
    HB = H // HSPLIT
    out = pl.pallas_call(
        functools.partial(_fused_kernel, B=B, C=C, K=K, N=N, HB=HB),
        grid=(B, HSPLIT),
        in_specs=[
            pl.BlockSpec((1, 1, C), lambda b, h: (b, 0, 0)),
            pl.BlockSpec((1, 1, C), lambda b, h: (b, 0, 0)),
            pl.BlockSpec((1, L, HB), lambda b, h: (b, 0, h)),
            pl.BlockSpec((N, H), lambda b, h: (0, 0)),
            pl.BlockSpec((N, H), lambda b, h: (0, 0)),
            pl.BlockSpec((1, 1, N), lambda b, h: (0, 0, 0)),
            pl.BlockSpec((1, N, K), lambda b, h: (0, 0, 0)),
            pl.BlockSpec((1, N, K), lambda b, h: (0, 0, 0)),
            pl.BlockSpec((1, 1, N), lambda b, h: (0, 0, 0)),
            pl.BlockSpec((1, 1, N), lambda b, h: (0, 0, 0)),
        ],
        out_specs=pl.BlockSpec((1, 1), lambda b, h: (0, 0)),
        out_shape=jax.ShapeDtypeStruct((1, 1), jnp.float32),
        scratch_shapes=[
            pltpu.VMEM((N, H), jnp.float32),
        ],
    )(starts, lims, code_hidden, comment_centers, code_centers, c2c, nb, ns,
      starts.reshape(1, 1, N), lims.reshape(1, 1, N))

    return out[0, 0]


# speedup vs baseline: 1.0056x; 1.0056x over previous
import functools

import jax
import jax.numpy as jnp
from jax.experimental import pallas as pl
from jax.experimental.pallas import tpu as pltpu

TEMPERATURE = 0.1


def _fused_kernel(starts_ref, lims_ref, ch_ref, cc_ref, codec_ref, c2c_ref,
                  nb_ref, ns_ref, sall_ref, lall_ref, out_ref, pooled_ref,
                  *, B, C, K, N, HB):
    b = pl.program_id(0)
    l = pl.program_id(1)
    nl = pl.num_programs(1)
    s = starts_ref[0, 0, :]
    lim = lims_ref[0, 0, :]
    LB = ch_ref.shape[1]
    t = jax.lax.broadcasted_iota(jnp.int32, (C, LB), 1) + l * LB
    mask = (t >= s[:, None]) & (t <= lim[:, None])
    maskf = mask.astype(jnp.float32)
    part = jnp.dot(maskf, ch_ref[0], preferred_element_type=jnp.float32)

    @pl.when(l == 0)
    def _init():
        pooled_ref[pl.ds(b * C, C), :] = part

    @pl.when(l != 0)
    def _acc():
        pooled_ref[pl.ds(b * C, C), :] += part

    @pl.when((b == B - 1) & (l == nl - 1))
    def _epilogue():
        eps = jnp.float32(1e-12)
        cc = cc_ref[...]
        cc = cc / jnp.maximum(
            jnp.sqrt(jnp.sum(cc * cc, axis=1, keepdims=True)), eps)
        codec = codec_ref[...]
        codec = codec / jnp.maximum(
            jnp.sqrt(jnp.sum(codec * codec, axis=1, keepdims=True)), eps)

        c2c = c2c_ref[0, 0, :]
        c2c_cl = jnp.clip(c2c, 0, N - 1)
        jj = jax.lax.broadcasted_iota(jnp.int32, (N, N), 1)
        sel_pos = (jj == c2c_cl[:, None]).astype(jnp.float32)
        code_cent = jnp.dot(sel_pos, codec,
                            preferred_element_type=jnp.float32)
        pos_sim = jnp.sum(cc * code_cent, axis=1)

        cnt = jnp.maximum(
            lall_ref[0, 0, :] - sall_ref[0, 0, :] + 1, 0
        ).astype(jnp.float32)
        pooled = pooled_ref[...]
        pooled = pooled / jnp.maximum(cnt, 1.0)[:, None]
        pooled = pooled / jnp.maximum(
            jnp.sqrt(jnp.sum(pooled * pooled, axis=1, keepdims=True)), eps)
        S = jnp.dot(cc, pooled.T,
                    preferred_element_type=jnp.float32)

        nb = nb_ref[0, :, :]
        ns = ns_ref[0, :, :]
        in_range = (nb < B) & (ns < C)
        j = jnp.clip(nb, 0, B - 1) * C + jnp.clip(ns, 0, C - 1)
        jk = jax.lax.broadcasted_iota(jnp.int32, (N, K, N), 2)
        sel = (jk == j[:, :, None]).astype(jnp.float32)
        E = jnp.sum(S[:, None, :] * sel, axis=2)
        cnt_pos = (cnt > 0.0).astype(jnp.float32)
        neg_has = jnp.sum(cnt_pos[None, None, :] * sel, axis=2) > 0.0
        vmask = in_range & neg_has

        neg_exp = jnp.exp(E / TEMPERATURE)
        neg_sum = jnp.sum(jnp.where(vmask, neg_exp, 0.0), axis=1)
        pos_exp = jnp.exp(pos_sim / TEMPERATURE)
        lv = -jnp.log(pos_exp / (pos_exp + neg_sum + 1e-08))
        valid = (c2c < N) & jnp.any(vmask, axis=1)
        vals = jnp.where(valid, lv, 0.0)
        total = jnp.sum(vals)
        n = jnp.sum(valid.astype(jnp.float32))
        res = jnp.where(n > 0.0, total / jnp.maximum(n, 1.0), 0.0)
        out_ref[...] = jnp.reshape(res, (1, 1))


@jax.jit
def kernel(comment_centers, code_centers, all_code_centers,
           comment_to_code_map, negative_sample_indices, nl_hidden,
           code_hidden, total_code_tokens_list, valid_code_spans_batch,
           valid_comment_spans_batch, step_descriptions_batch):
    del all_code_centers, nl_hidden, valid_comment_spans_batch
    del step_descriptions_batch
    B, L, H = code_hidden.shape
    N, _ = comment_centers.shape
    _, C, K, _ = negative_sample_indices.shape

    spans = valid_code_spans_batch.astype(jnp.int32)
    starts = spans[:, :, 1, 0].reshape(B, 1, C)
    totals = total_code_tokens_list.astype(jnp.int32)
    lims = jnp.minimum(spans[:, :, 1, 1],
                       totals[:, None]).reshape(B, 1, C)

    negs = negative_sample_indices.astype(jnp.int32).reshape(N, K, 2)
    nb = negs[:, :, 0].reshape(1, N, K)
    ns = negs[:, :, 1].reshape(1, N, K)
    c2c = comment_to_code_map.astype(jnp.int32).reshape(1, 1, N)

    LSPLIT = 2
    LB = L // LSPLIT
    out = pl.pallas_call(
        functools.partial(_fused_kernel, B=B, C=C, K=K, N=N, HB=H),
        grid=(B, LSPLIT),
        in_specs=[
            pl.BlockSpec((1, 1, C), lambda b, h: (b, 0, 0)),
            pl.BlockSpec((1, 1, C), lambda b, h: (b, 0, 0)),
            pl.BlockSpec((1, LB, H), lambda b, h: (b, h, 0)),
            pl.BlockSpec((N, H), lambda b, h: (0, 0)),
            pl.BlockSpec((N, H), lambda b, h: (0, 0)),
            pl.BlockSpec((1, 1, N), lambda b, h: (0, 0, 0)),
            pl.BlockSpec((1, N, K), lambda b, h: (0, 0, 0)),
            pl.BlockSpec((1, N, K), lambda b, h: (0, 0, 0)),
            pl.BlockSpec((1, 1, N), lambda b, h: (0, 0, 0)),
            pl.BlockSpec((1, 1, N), lambda b, h: (0, 0, 0)),
        ],
        out_specs=pl.BlockSpec((1, 1), lambda b, h: (0, 0)),
        out_shape=jax.ShapeDtypeStruct((1, 1), jnp.float32),
        scratch_shapes=[
            pltpu.VMEM((N, H), jnp.float32),
        ],
    )(starts, lims, code_hidden, comment_centers, code_centers, c2c, nb, ns,
      starts.reshape(1, 1, N), lims.reshape(1, 1, N))

    return out[0, 0]
